# baseline (speedup 1.0000x reference)
import functools

import numpy as np

import jax
import jax.numpy as jnp
from jax import lax
from jax.experimental import pallas as pl
from jax.experimental.pallas import tpu as pltpu
from jax.experimental.pallas import tpu_sc as plsc

VOCAB = 1000000
N_EMBED = 64
BATCH = 16384
CTX = 20

NC = 2
NS = 16
NW = NC * NS

LANES = 16
D_VECS = N_EMBED // LANES
PAIR_W = 2 * N_EMBED

B_PER_W = BATCH // NW
IDX_PER_W = B_PER_W * CTX
GCHUNK = 128
N_GROWS = IDX_PER_W // GCHUNK
B_CHUNK = 32
I_CHUNK = B_CHUNK * CTX
G_PER_STEP = I_CHUNK // GCHUNK
N_STEPS = B_PER_W // B_CHUNK

def _splat(vec, lane):
    return lax.gather(
        vec,
        lax.reshape(lax.broadcast(jnp.int32(lane), (LANES,)), (LANES, 1)),
        lax.GatherDimensionNumbers(
            offset_dims=(), collapsed_slice_dims=(0,), start_index_map=(0,)
        ),
        (1,),
        mode=lax.GatherScatterMode.PROMISE_IN_BOUNDS,
    )


def _sc_body(table_hbm, xr_hbm, out_hbm, idx_v, row_v, pofs_v, rows_v, out_v,
             sem):
    wid = lax.axis_index("s") * NC + lax.axis_index("c")
    pltpu.sync_copy(xr_hbm.at[wid], idx_v)

    inv_ctx = jnp.float32(1.0 / CTX)
    iota = lax.iota(jnp.int32, LANES)

    for g in range(N_STEPS):
        def prep_body(k, carry):
            j = k >> 3
            c8 = lax.bitwise_and(k, 7)
            v = idx_v[g * G_PER_STEP + j, pl.ds(c8 * LANES, LANES)]
            row_v[j, pl.ds(c8 * LANES, LANES)] = lax.shift_right_logical(v, 1)
            pofs_v[j, pl.ds(c8 * LANES, LANES)] = lax.shift_left(
                lax.bitwise_and(v, 1), 6
            )
            return carry

        lax.fori_loop(0, G_PER_STEP * (GCHUNK // LANES), prep_body, 0)

        copies = [
            pltpu.async_copy(
                table_hbm.at[row_v.at[j]],
                rows_v.at[pl.ds(j * GCHUNK, GCHUNK)],
                sem,
            )
            for j in range(G_PER_STEP)
        ]
        for c in copies:
            c.wait()

        def acc_body(b, carry):
            r0 = b * CTX
            p_lo = r0 + iota
            p_hi = p_lo + 4
            ofs_lo = plsc.load_gather(
                pofs_v,
                [lax.shift_right_logical(p_lo, 7), lax.bitwise_and(p_lo, 127)],
            )
            ofs_hi = plsc.load_gather(
                pofs_v,
                [lax.shift_right_logical(p_hi, 7), lax.bitwise_and(p_hi, 127)],
            )
            accs = [None] * D_VECS
            for c in range(CTX):
                if c < LANES:
                    half = _splat(ofs_lo, c)
                else:
                    half = _splat(ofs_hi, c - 4)
                rvec = jnp.full((LANES,), r0 + c, jnp.int32)
                for d in range(D_VECS):
                    piece = plsc.load_gather(
                        rows_v, [rvec, half + (iota + d * LANES)]
                    )
                    accs[d] = piece if c == 0 else accs[d] + piece
            for d in range(D_VECS):
                out_v[b, pl.ds(LANES * d, LANES)] = accs[d] * inv_ctx
            return carry

        lax.fori_loop(0, B_CHUNK, acc_body, 0)
        pltpu.sync_copy(
            out_v, out_hbm.at[pl.ds(wid * B_PER_W + g * B_CHUNK, B_CHUNK)]
        )


@jax.jit
def _cbow_mean(x, table):
    xr = x.reshape(NW, N_GROWS, GCHUNK).astype(jnp.int32)
    tpairs = table.reshape(VOCAB // 2, PAIR_W)
    mesh = plsc.VectorSubcoreMesh(core_axis_name="c", subcore_axis_name="s")
    k = pl.kernel(
        _sc_body,
        out_type=jax.ShapeDtypeStruct((BATCH, N_EMBED), jnp.float32),
        mesh=mesh,
        scratch_types=[
            pltpu.VMEM((N_GROWS, GCHUNK), jnp.int32),
            pltpu.VMEM((G_PER_STEP, GCHUNK), jnp.int32),
            pltpu.VMEM((G_PER_STEP, GCHUNK), jnp.int32),
            pltpu.VMEM((I_CHUNK, PAIR_W), jnp.float32),
            pltpu.VMEM((B_CHUNK, N_EMBED), jnp.float32),
            pltpu.SemaphoreType.DMA,
        ],
        compiler_params=pltpu.CompilerParams(needs_layout_passes=False),
    )
    return k(tpairs, xr)


def kernel(x, y, table):
    del y
    return _cbow_mean(x, table)


# speedup vs baseline: 1.0986x; 1.0986x over previous
import functools

import numpy as np

import jax
import jax.numpy as jnp
from jax import lax
from jax.experimental import pallas as pl
from jax.experimental.pallas import tpu as pltpu
from jax.experimental.pallas import tpu_sc as plsc

VOCAB = 1000000
N_EMBED = 64
BATCH = 16384
CTX = 20

NC = 2
NS = 16
NW = NC * NS

LANES = 16
D_VECS = N_EMBED // LANES

B_PER_W = BATCH // NW
B_SUB = 64
N_SUBS = B_PER_W // B_SUB


def _sc_body(table_hbm, xt_hbm, out_hbm, idx_v, rows_v, out_v, sem):
    wid = lax.axis_index("s") * NC + lax.axis_index("c")
    base = wid * B_PER_W
    pltpu.sync_copy(xt_hbm.at[:, pl.ds(base, B_PER_W)], idx_v)

    inv_ctx = jnp.float32(1.0 / CTX)

    for s in range(N_SUBS):
        copies = [
            pltpu.async_copy(
                table_hbm.at[idx_v.at[c, pl.ds(s * B_SUB, B_SUB)]],
                rows_v.at[c],
                sem,
            )
            for c in range(CTX)
        ]
        for cp in copies:
            cp.wait()

        def acc_body(b, carry):
            for d in range(D_VECS):
                acc = rows_v[0, b, pl.ds(LANES * d, LANES)]
                for c in range(1, CTX):
                    acc = acc + rows_v[c, b, pl.ds(LANES * d, LANES)]
                out_v[b, pl.ds(LANES * d, LANES)] = acc * inv_ctx
            return carry

        lax.fori_loop(0, B_SUB, acc_body, 0)
        pltpu.sync_copy(out_v, out_hbm.at[pl.ds(base + s * B_SUB, B_SUB)])


@jax.jit
def _cbow_mean(x, table):
    xt = x.T.astype(jnp.int32)
    mesh = plsc.VectorSubcoreMesh(core_axis_name="c", subcore_axis_name="s")
    k = pl.kernel(
        _sc_body,
        out_type=jax.ShapeDtypeStruct((BATCH, N_EMBED), jnp.float32),
        mesh=mesh,
        scratch_types=[
            pltpu.VMEM((CTX, B_PER_W), jnp.int32),
            pltpu.VMEM((CTX, B_SUB, N_EMBED), jnp.float32),
            pltpu.VMEM((B_SUB, N_EMBED), jnp.float32),
            pltpu.SemaphoreType.DMA,
        ],
        compiler_params=pltpu.CompilerParams(use_tc_tiling_on_sc=False),
    )
    return k(table, xt)


def kernel(x, y, table):
    del y
    return _cbow_mean(x, table)
